# Initial kernel scaffold; baseline (speedup 1.0000x reference)
import jax
import jax.numpy as jnp
from jax.experimental import pallas as pl


def kernel(z, pos, batch, atom_emb, W_send, W_recv, fw1, fb1, fw2, fb2, gw1, gb1, gw2, gb2, ln_g, ln_b, r_w1, r_b1, r_w2, r_b2):
    raise NotImplementedError("write your pallas kernel here")


# speedup vs baseline: 23.0582x; 23.0582x over previous
import functools
import math

import jax
import jax.numpy as jnp
from jax import lax
from jax.experimental import pallas as pl
from jax.experimental.pallas import tpu as pltpu

BR = 8
BC = 128
CUTOFF = 5.0
N_MOL = 500
LN_EPS = 1e-5


def _embed_kernel(z_ref, emb_ref, x_ref):
    i = pl.program_id(0)
    r0 = i * 128
    zr = z_ref[pl.ds(r0, 128), :]
    mz = emb_ref.shape[0]
    onehot = (zr == lax.broadcasted_iota(jnp.int32, (128, mz), 1)).astype(jnp.float32)
    x_ref[pl.ds(r0, 128), :] = jnp.dot(onehot, emb_ref[...],
                                       preferred_element_type=jnp.float32,
                                       precision=lax.Precision.HIGHEST)


def _layer_kernel(lo_ref, hi_ref,
                  x_ref, posr_ref, post_ref, brow_ref, bcol_ref,
                  ws_ref, wr_ref, fw1_ref, fb1_ref, fw2_ref, fb2_ref,
                  gw1t_ref, gb1_ref, gw2t_ref, gb2_ref, lng_ref, lnb_ref,
                  offs_ref, xout_ref, *, coeff, n_rbf, d_stalk):
    i = pl.program_id(0)
    r0 = i * BR
    lo = lo_ref[i]
    hi = hi_ref[i]
    c_start = (lo // BC) * BC
    n_t = (hi - c_start + BC - 1) // BC

    pos_r = posr_ref[pl.ds(r0, BR), :]
    b_r = brow_ref[pl.ds(r0, BR), :]
    ridx = r0 + lax.broadcasted_iota(jnp.int32, (BR, BC), 0)
    offs = offs_ref[...]

    def tile(t, qacc):
        c0 = c_start + t * BC
        xc = x_ref[pl.ds(c0, BC), :]
        projT = lax.dot_general(ws_ref[...], xc, (((1,), (1,)), ((), ())),
                                preferred_element_type=jnp.float32)
        d2 = jnp.zeros((BR, BC), jnp.float32)
        for k in range(3):
            dr = pos_r[:, k:k + 1] - post_ref[k:k + 1, pl.ds(c0, BC)]
            d2 = d2 + dr * dr
        dist = jnp.sqrt(d2)
        b_c = bcol_ref[0:1, pl.ds(c0, BC)]
        cidx = c0 + lax.broadcasted_iota(jnp.int32, (BR, BC), 1)
        mask = (b_r == b_c) & (dist < CUTOFF) & (ridx != cidx)
        env = 0.5 * (1.0 + jnp.cos((math.pi / CUTOFF) * dist))
        w = jnp.where(mask, env, 0.0)
        out = []
        for r in range(BR):
            delta = dist[r:r + 1, :] - offs
            rbf = jnp.exp(coeff * delta * delta)
            y = jnp.dot(fw1_ref[...], rbf,
                        preferred_element_type=jnp.float32) + fb1_ref[...]
            y = jax.nn.silu(y)
            phi = jnp.dot(fw2_ref[...], y,
                          preferred_element_type=jnp.float32) + fb2_ref[...]
            out.append(qacc[r] + phi * w[r:r + 1, :] * projT)
        return tuple(out)

    qacc0 = tuple(jnp.zeros((d_stalk, BC), jnp.float32) for _ in range(BR))
    qacc = lax.fori_loop(0, n_t, tile, qacc0)
    stalk = jnp.concatenate([jnp.sum(qacc[r], axis=1, keepdims=True)
                             for r in range(BR)], axis=1)
    agg = lax.dot_general(stalk, wr_ref[...], (((0,), (0,)), ((), ())),
                          preferred_element_type=jnp.float32)

    g = jnp.dot(agg, gw1t_ref[...], preferred_element_type=jnp.float32) + gb1_ref[...]
    g = jax.nn.silu(g)
    g = jnp.dot(g, gw2t_ref[...], preferred_element_type=jnp.float32) + gb2_ref[...]
    h = x_ref[pl.ds(r0, BR), :] + g
    mu = jnp.mean(h, axis=1, keepdims=True)
    var = jnp.mean((h - mu) ** 2, axis=1, keepdims=True)
    xout_ref[pl.ds(r0, BR), :] = (h - mu) / jnp.sqrt(var + LN_EPS) * lng_ref[...] + lnb_ref[...]


def _readout_kernel(x_ref, bcol_ref, rw1t_ref, rb1_ref, rw2t_ref, rb2_ref, e_ref):
    i = pl.program_id(0)
    r0 = i * 128
    xb = x_ref[pl.ds(r0, 128), :]
    y = jnp.dot(xb, rw1t_ref[...], preferred_element_type=jnp.float32) + rb1_ref[...]
    y = jax.nn.silu(y)
    ae = jnp.dot(y, rw2t_ref[...], preferred_element_type=jnp.float32) + rb2_ref[...]
    b_r = bcol_ref[0:1, pl.ds(r0, 128)]
    n_mol = e_ref.shape[1]
    sel = b_r.reshape(128, 1) == lax.broadcasted_iota(jnp.int32, (1, n_mol), 1)
    contrib = jnp.sum(jnp.where(sel, ae, 0.0), axis=0, keepdims=True)

    @pl.when(i == 0)
    def _():
        e_ref[...] = jnp.zeros_like(e_ref)

    e_ref[...] += contrib


def _full_spec(shape):
    nd = len(shape)
    return pl.BlockSpec(shape, lambda i, lo, hi: (0,) * nd)


def kernel(z, pos, batch, atom_emb, W_send, W_recv, fw1, fb1, fw2, fb2,
           gw1, gb1, gw2, gb2, ln_g, ln_b, r_w1, r_b1, r_w2, r_b2):
    n = z.shape[0]
    n_layers, d_stalk, d_node = W_send.shape
    n_rbf = fw1.shape[-1]
    n_pad = ((n + BC - 1) // BC) * BC
    nb = n_pad // BR
    nb_real = n // BR

    f32 = jnp.float32
    z32 = z.astype(jnp.int32)
    batch32 = batch.astype(jnp.int32)

    zp = jnp.pad(z32, (0, n_pad - n)).reshape(n_pad, 1)
    bp = jnp.pad(batch32, (0, n_pad - n), constant_values=-1)
    brow = bp.reshape(n_pad, 1)
    bcol = bp.reshape(1, n_pad)
    posr = jnp.pad(pos.astype(f32), ((0, n_pad - n), (0, 1)),
                   constant_values=1e6)
    post = posr[:, :3].T

    b_first = batch32[0::BR]
    b_last = batch32[BR - 1::BR]
    lo = jnp.searchsorted(batch32, b_first, side='left').astype(jnp.int32)
    hi = jnp.searchsorted(batch32, b_last, side='right').astype(jnp.int32)
    row_lo = jnp.pad(lo, (0, nb - nb_real))
    row_hi = jnp.pad(hi, (0, nb - nb_real))

    offset = jnp.linspace(0.0, CUTOFF, n_rbf, dtype=f32).reshape(1, n_rbf)
    coeff = -0.5 / float(CUTOFF / (n_rbf - 1)) ** 2

    x = pl.pallas_call(
        _embed_kernel,
        grid=(n_pad // 128,),
        in_specs=[pl.BlockSpec(zp.shape, lambda i: (0, 0)),
                  pl.BlockSpec(atom_emb.shape, lambda i: (0, 0))],
        out_specs=pl.BlockSpec((n_pad, d_node), lambda i: (0, 0)),
        out_shape=jax.ShapeDtypeStruct((n_pad, d_node), f32),
    )(zp, atom_emb.astype(f32))

    layer_fn = functools.partial(_layer_kernel, coeff=coeff, n_rbf=n_rbf,
                                 d_stalk=d_stalk)
    offs3 = jnp.broadcast_to(offset.reshape(n_rbf, 1), (n_rbf, BC))
    for l in range(n_layers):
        fb1_b = jnp.broadcast_to(fb1[l].reshape(d_node, 1), (d_node, BC))
        fb2_b = jnp.broadcast_to(fb2[l].reshape(d_stalk, 1), (d_stalk, BC))
        args = (x, posr, post, brow, bcol,
                W_send[l], W_recv[l], fw1[l], fb1_b,
                fw2[l], fb2_b,
                gw1[l].T, gb1[l].reshape(1, d_node),
                gw2[l].T, gb2[l].reshape(1, d_node),
                ln_g[l].reshape(1, d_node), ln_b[l].reshape(1, d_node), offs3)
        grid_spec = pltpu.PrefetchScalarGridSpec(
            num_scalar_prefetch=2,
            grid=(nb,),
            in_specs=[_full_spec(a.shape) for a in args],
            out_specs=_full_spec((n_pad, d_node)),
        )
        x = pl.pallas_call(
            layer_fn,
            grid_spec=grid_spec,
            out_shape=jax.ShapeDtypeStruct((n_pad, d_node), f32),
        )(row_lo, row_hi, *args)

    d_half = r_w1.shape[0]
    energy = pl.pallas_call(
        _readout_kernel,
        grid=(n_pad // 128,),
        in_specs=[pl.BlockSpec((n_pad, d_node), lambda i: (0, 0)),
                  pl.BlockSpec(bcol.shape, lambda i: (0, 0)),
                  pl.BlockSpec((d_node, d_half), lambda i: (0, 0)),
                  pl.BlockSpec((1, d_half), lambda i: (0, 0)),
                  pl.BlockSpec((d_half, 1), lambda i: (0, 0)),
                  pl.BlockSpec((1, 1), lambda i: (0, 0))],
        out_specs=pl.BlockSpec((1, N_MOL), lambda i: (0, 0)),
        out_shape=jax.ShapeDtypeStruct((1, N_MOL), f32),
    )(x, bcol, r_w1.T, r_b1.reshape(1, d_half), r_w2.T, r_b2.reshape(1, 1))

    return energy.reshape(N_MOL)
